# baseline (device time: 26085 ns/iter reference)
import jax
import jax.numpy as jnp
from jax import lax
from jax.experimental import pallas as pl
from jax.experimental.pallas import tpu as pltpu

N_DEV = 4


def _gelu(y):
    c = 0.7978845608028654
    return 0.5 * y * (1.0 + jnp.tanh(c * (y + 0.044715 * y * y * y)))


def kernel(x, w_mat):
    m_per, k = x.shape
    _, n = w_mat.shape
    n_per = n // N_DEV
    n_half = n_per // 2

    pieces = [
        (1, 0, "a", 0),
        (1, n_half, "a", 1),
        (2, 0, "b", 0),
        (3, 0, "b", 1),
    ]

    def body(
        x_ref, w_hbm, out_ref,
        wbuf_a, wbuf_b, send_a, send_b, recv_a, recv_b,
        copy_sems, send_sems, recv_sems,
    ):
        my_pos = lax.axis_index("i")

        copies = []
        for s, (d, off, kind, slot) in enumerate(pieces):
            col = ((my_pos + d) % N_DEV) * n_per + off
            width_buf = wbuf_a if kind == "a" else wbuf_b
            w_slice = n_half if kind == "a" else n_per
            cp = pltpu.make_async_copy(
                w_hbm.at[:, pl.ds(col, w_slice)],
                width_buf.at[slot],
                copy_sems.at[s],
            )
            cp.start()
            copies.append(cp)
        own_cp = pltpu.make_async_copy(
            w_hbm.at[:, pl.ds(my_pos * n_per, n_per)],
            wbuf_b.at[2],
            copy_sems.at[4],
        )
        own_cp.start()

        barrier_sem = pltpu.get_barrier_semaphore()
        for d in range(1, N_DEV):
            pl.semaphore_signal(
                barrier_sem,
                inc=1,
                device_id=((my_pos + d) % N_DEV,),
                device_id_type=pl.DeviceIdType.MESH,
            )
        x_bf = x_ref[:, :].astype(jnp.bfloat16)
        pl.semaphore_wait(barrier_sem, N_DEV - 1)

        rdmas = []
        for s, (d, off, kind, slot) in enumerate(pieces):
            tgt = (my_pos + d) % N_DEV
            copies[s].wait()
            if kind == "a":
                wj = wbuf_a[slot, :, :].astype(jnp.bfloat16)
                src_buf, dst_buf = send_a, recv_a
            else:
                wj = wbuf_b[slot, :, :].astype(jnp.bfloat16)
                src_buf, dst_buf = send_b, recv_b
            y = jnp.dot(x_bf, wj, preferred_element_type=jnp.float32)
            if kind == "b":
                y = _gelu(y)
            src_buf[slot, :, :] = y.astype(jnp.bfloat16)
            rdma = pltpu.make_async_remote_copy(
                src_ref=src_buf.at[slot],
                dst_ref=dst_buf.at[slot],
                send_sem=send_sems.at[s],
                recv_sem=recv_sems.at[s],
                device_id=(tgt,),
                device_id_type=pl.DeviceIdType.MESH,
            )
            rdma.start()
            rdmas.append(rdma)

        own_cp.wait()
        wj = wbuf_b[2, :, :].astype(jnp.bfloat16)
        y = jnp.dot(x_bf, wj, preferred_element_type=jnp.float32)
        out_ref[pl.ds(my_pos * m_per, m_per), :] = _gelu(y)

        for s, (d, off, kind, slot) in enumerate(pieces):
            src = (my_pos - d) % N_DEV
            rdmas[s].wait_recv()
            if kind == "a":
                yin = recv_a[slot, :, :].astype(jnp.float32)
                out_ref[
                    pl.ds(src * m_per, m_per), pl.ds(off, n_half)
                ] = _gelu(yin)
            else:
                out_ref[pl.ds(src * m_per, m_per), :] = recv_b[
                    slot, :, :
                ].astype(jnp.float32)

        for rdma in rdmas:
            rdma.wait_send()

    return pl.pallas_call(
        body,
        out_shape=jax.ShapeDtypeStruct((N_DEV * m_per, n_per), jnp.float32),
        in_specs=[
            pl.BlockSpec(memory_space=pltpu.VMEM),
            pl.BlockSpec(memory_space=pl.ANY),
        ],
        out_specs=pl.BlockSpec(memory_space=pltpu.VMEM),
        scratch_shapes=[
            pltpu.VMEM((2, k, n_half), jnp.float32),
            pltpu.VMEM((3, k, n_per), jnp.float32),
            pltpu.VMEM((2, m_per, n_half), jnp.bfloat16),
            pltpu.VMEM((2, m_per, n_per), jnp.bfloat16),
            pltpu.VMEM((2, m_per, n_half), jnp.bfloat16),
            pltpu.VMEM((2, m_per, n_per), jnp.bfloat16),
            pltpu.SemaphoreType.DMA((5,)),
            pltpu.SemaphoreType.DMA((4,)),
            pltpu.SemaphoreType.DMA((4,)),
        ],
        compiler_params=pltpu.CompilerParams(collective_id=0),
    )(x, w_mat)


# device time: 25894 ns/iter; 1.0074x vs baseline; 1.0074x over previous
import jax
import jax.numpy as jnp
from jax import lax
from jax.experimental import pallas as pl
from jax.experimental.pallas import tpu as pltpu

N_DEV = 4
_HOP_ORDER = (1, 2, 3)
_HALVES = 2


def _gelu(y):
    c = 0.7978845608028654
    return 0.5 * y * (1.0 + jnp.tanh(c * (y + 0.044715 * y * y * y)))


def kernel(x, w_mat):
    m_per, k = x.shape
    _, n = w_mat.shape
    n_per = n // N_DEV
    n_sub = n_per // _HALVES

    remote_subs = [(d, h) for d in _HOP_ORDER for h in range(_HALVES)]

    def body(
        x_ref, w_hbm, out_ref, wbuf, send_buf, recv_buf,
        copy_sems, send_sems, recv_sems,
    ):
        my_pos = lax.axis_index("i")

        copies = []
        for s, (d, h) in enumerate(remote_subs + [(0, 0), (0, 1)]):
            col = ((my_pos + d) % N_DEV) * n_per + h * n_sub
            cp = pltpu.make_async_copy(
                w_hbm.at[:, pl.ds(col, n_sub)], wbuf.at[s], copy_sems.at[s]
            )
            cp.start()
            copies.append(cp)

        barrier_sem = pltpu.get_barrier_semaphore()
        for d in range(1, N_DEV):
            pl.semaphore_signal(
                barrier_sem,
                inc=1,
                device_id=((my_pos + d) % N_DEV,),
                device_id_type=pl.DeviceIdType.MESH,
            )
        pl.semaphore_wait(barrier_sem, N_DEV - 1)

        x_bf = x_ref[:, :].astype(jnp.bfloat16)

        rdmas = []
        for s, (d, h) in enumerate(remote_subs):
            tgt = (my_pos + d) % N_DEV
            copies[s].wait()
            wj = wbuf[s, :, :].astype(jnp.bfloat16)
            y = jnp.dot(x_bf, wj, preferred_element_type=jnp.float32)
            send_buf[s, :, :] = y.astype(jnp.bfloat16)
            rdma = pltpu.make_async_remote_copy(
                src_ref=send_buf.at[s],
                dst_ref=recv_buf.at[s],
                send_sem=send_sems.at[s],
                recv_sem=recv_sems.at[s],
                device_id=(tgt,),
                device_id_type=pl.DeviceIdType.MESH,
            )
            rdma.start()
            rdmas.append(rdma)

        for h in range(_HALVES):
            copies[6 + h].wait()
            wj = wbuf[6 + h, :, :].astype(jnp.bfloat16)
            y = jnp.dot(x_bf, wj, preferred_element_type=jnp.float32)
            out_ref[pl.ds(my_pos * m_per, m_per), pl.ds(h * n_sub, n_sub)] = (
                _gelu(y)
            )

        for s, (d, h) in enumerate(remote_subs):
            src = (my_pos - d) % N_DEV
            rdmas[s].wait_recv()
            yin = recv_buf[s, :, :].astype(jnp.float32)
            out_ref[pl.ds(src * m_per, m_per), pl.ds(h * n_sub, n_sub)] = (
                _gelu(yin)
            )

        for rdma in rdmas:
            rdma.wait_send()

    n_remote = len(remote_subs)
    return pl.pallas_call(
        body,
        out_shape=jax.ShapeDtypeStruct((N_DEV * m_per, n_per), jnp.float32),
        in_specs=[
            pl.BlockSpec(memory_space=pltpu.VMEM),
            pl.BlockSpec(memory_space=pl.ANY),
        ],
        out_specs=pl.BlockSpec(memory_space=pltpu.VMEM),
        scratch_shapes=[
            pltpu.VMEM((n_remote + 2, k, n_sub), jnp.float32),
            pltpu.VMEM((n_remote, m_per, n_sub), jnp.bfloat16),
            pltpu.VMEM((n_remote, m_per, n_sub), jnp.bfloat16),
            pltpu.SemaphoreType.DMA((n_remote + 2,)),
            pltpu.SemaphoreType.DMA((n_remote,)),
            pltpu.SemaphoreType.DMA((n_remote,)),
        ],
        compiler_params=pltpu.CompilerParams(collective_id=0),
    )(x, w_mat)


# device time: 23309 ns/iter; 1.1191x vs baseline; 1.1109x over previous
import jax
import jax.numpy as jnp
from jax import lax
from jax.experimental import pallas as pl
from jax.experimental.pallas import tpu as pltpu

N_DEV = 4
_HOP_ORDER = (2, 1, 3)
_HALVES = 2


def _gelu(y):
    c = 0.7978845608028654
    return 0.5 * y * (1.0 + jnp.tanh(c * (y + 0.044715 * y * y * y)))


def kernel(x, w_mat):
    m_per, k = x.shape
    _, n = w_mat.shape
    n_per = n // N_DEV
    n_sub = n_per // _HALVES

    remote_subs = [(d, h) for d in _HOP_ORDER for h in range(_HALVES)]

    def body(
        x_ref, w_hbm, out_ref, wbuf, send_buf, recv_buf,
        copy_sems, send_sems, recv_sems,
    ):
        my_pos = lax.axis_index("i")

        copies = []
        for s, (d, h) in enumerate(remote_subs + [(0, 0), (0, 1)]):
            col = ((my_pos + d) % N_DEV) * n_per + h * n_sub
            cp = pltpu.make_async_copy(
                w_hbm.at[:, pl.ds(col, n_sub)], wbuf.at[s], copy_sems.at[s]
            )
            cp.start()
            copies.append(cp)

        barrier_sem = pltpu.get_barrier_semaphore()
        for d in range(1, N_DEV):
            pl.semaphore_signal(
                barrier_sem,
                inc=1,
                device_id=((my_pos + d) % N_DEV,),
                device_id_type=pl.DeviceIdType.MESH,
            )
        pl.semaphore_wait(barrier_sem, N_DEV - 1)

        x_bf = x_ref[:, :].astype(jnp.bfloat16)

        rdmas = []
        for s, (d, h) in enumerate(remote_subs):
            tgt = (my_pos + d) % N_DEV
            copies[s].wait()
            wj = wbuf[s, :, :].astype(jnp.bfloat16)
            y = jnp.dot(x_bf, wj, preferred_element_type=jnp.float32)
            send_buf[s, :, :] = y.astype(jnp.bfloat16)
            rdma = pltpu.make_async_remote_copy(
                src_ref=send_buf.at[s],
                dst_ref=recv_buf.at[s],
                send_sem=send_sems.at[s],
                recv_sem=recv_sems.at[s],
                device_id=(tgt,),
                device_id_type=pl.DeviceIdType.MESH,
            )
            rdma.start()
            rdmas.append(rdma)

        for h in range(_HALVES):
            copies[6 + h].wait()
            wj = wbuf[6 + h, :, :].astype(jnp.bfloat16)
            y = jnp.dot(x_bf, wj, preferred_element_type=jnp.float32)
            out_ref[pl.ds(my_pos * m_per, m_per), pl.ds(h * n_sub, n_sub)] = (
                _gelu(y)
            )

        for s, (d, h) in enumerate(remote_subs):
            src = (my_pos - d) % N_DEV
            rdmas[s].wait_recv()
            yin = recv_buf[s, :, :].astype(jnp.float32)
            out_ref[pl.ds(src * m_per, m_per), pl.ds(h * n_sub, n_sub)] = (
                _gelu(yin)
            )

        for rdma in rdmas:
            rdma.wait_send()

    n_remote = len(remote_subs)
    return pl.pallas_call(
        body,
        out_shape=jax.ShapeDtypeStruct((N_DEV * m_per, n_per), jnp.float32),
        in_specs=[
            pl.BlockSpec(memory_space=pltpu.VMEM),
            pl.BlockSpec(memory_space=pl.ANY),
        ],
        out_specs=pl.BlockSpec(memory_space=pltpu.VMEM),
        scratch_shapes=[
            pltpu.VMEM((n_remote + 2, k, n_sub), jnp.float32),
            pltpu.VMEM((n_remote, m_per, n_sub), jnp.bfloat16),
            pltpu.VMEM((n_remote, m_per, n_sub), jnp.bfloat16),
            pltpu.SemaphoreType.DMA((n_remote + 2,)),
            pltpu.SemaphoreType.DMA((n_remote,)),
            pltpu.SemaphoreType.DMA((n_remote,)),
        ],
        compiler_params=pltpu.CompilerParams(collective_id=0),
    )(x, w_mat)
